# baseline (speedup 1.0000x reference)
import functools

import jax
import jax.numpy as jnp
from jax import lax
from jax.experimental import pallas as pl
from jax.experimental.pallas import tpu as pltpu
from jax.experimental.pallas import tpu_sc as plsc

_B, _N, _CODE_DIM, _K, _HIDDEN = 16, 1024, 256, 8192, 768
_COMMITMENT_COST = 0.25
_THRESHOLD = 0.5

_TOK = _B * _N


_info = plsc.get_sparse_core_info()
_NC, _NS = _info.num_cores, _info.num_subcores
_NW = _NC * _NS
_PER_W = _TOK // _NW
_CH = 128
_N_CH = _PER_W // _CH


def _make_sc_gather():
    mesh = plsc.VectorSubcoreMesh(core_axis_name="c", subcore_axis_name="s")

    @functools.partial(
        pl.kernel,
        mesh=mesh,
        out_type=jax.ShapeDtypeStruct((_TOK, _CODE_DIM), jnp.float32),
        scratch_types=[
            pltpu.VMEM((_N_CH, _CH), jnp.int32),
            pltpu.VMEM((_CH, _CODE_DIM), jnp.float32),
            pltpu.VMEM((_CH, _CODE_DIM), jnp.float32),
            pltpu.SemaphoreType.DMA,
            pltpu.SemaphoreType.DMA,
        ],
    )
    def sc_gather(table_hbm, idx_hbm, out_hbm, idx_v, rows0, rows1, sem0, sem1):
        wid = lax.axis_index("s") * _NC + lax.axis_index("c")
        base = wid * _PER_W
        pltpu.sync_copy(idx_hbm.at[wid], idx_v)
        bufs = (rows0, rows1)
        sems = (sem0, sem1)
        copies = [None, None]
        copies[0] = pltpu.async_copy(table_hbm.at[idx_v.at[0]], rows0, sem0)
        for c in range(_N_CH):
            cur = c % 2
            if c + 1 < _N_CH:
                nxt = (c + 1) % 2
                copies[nxt] = pltpu.async_copy(
                    table_hbm.at[idx_v.at[c + 1]], bufs[nxt], sems[nxt])
            copies[cur].wait()
            pltpu.sync_copy(bufs[cur], out_hbm.at[pl.ds(base + c * _CH, _CH)])

    return sc_gather


_sc_gather = _make_sc_gather()


_MBLK = 512


def _mm_body(q_ref, w_ref, b_ref, o_ref):
    o_ref[...] = lax.dot_general(
        q_ref[...], w_ref[...], (((1,), (1,)), ((), ())),
        preferred_element_type=jnp.float32) + b_ref[...]


_mm_call = pl.pallas_call(
    _mm_body,
    grid=(_TOK // _MBLK,),
    in_specs=[
        pl.BlockSpec((_MBLK, _CODE_DIM), lambda t: (t, 0)),
        pl.BlockSpec((_HIDDEN, _CODE_DIM), lambda t: (0, 0)),
        pl.BlockSpec((1, _HIDDEN), lambda t: (0, 0)),
    ],
    out_specs=pl.BlockSpec((_MBLK, _HIDDEN), lambda t: (t, 0)),
    out_shape=jax.ShapeDtypeStruct((_TOK, _HIDDEN), jnp.float32),
)


_SUB = _N // 128


def _stats_body(q_ref, m_ref, valid_ref, loss_ref, acc_ref):
    bidx = pl.program_id(0)
    q3 = q_ref[0].reshape(_SUB, 128, _CODE_DIM)
    m3 = m_ref[0].reshape(_SUB, 128, _CODE_DIM)
    sim = jnp.sum(q3 * m3, axis=2)
    sq = jnp.sum((m3 - q3) ** 2, axis=2)

    neg = jnp.float32(-jnp.inf)
    cur = jnp.float32(jnp.inf)
    removed = jnp.float32(0.0)
    for _ in range(5):
        mmax = jnp.max(jnp.where(sim < cur, sim, neg))
        cnt_eq = jnp.sum(jnp.where(sim == mmax, 1.0, 0.0))
        upd = removed < 5.0
        removed = jnp.where(upd, removed + cnt_eq, removed)
        cur = jnp.where(upd, mmax, cur)

    thresh = jnp.minimum(cur, jnp.float32(_THRESHOLD))
    validf = (sim >= thresh).astype(jnp.float32)
    valid_ref[0] = validf.astype(jnp.int32)

    num = jnp.sum(sq * validf)
    cnt = jnp.sum(validf)

    @pl.when(bidx == 0)
    def _init():
        acc_ref[0] = num
        acc_ref[1] = cnt

    @pl.when(bidx > 0)
    def _accum():
        acc_ref[0] = acc_ref[0] + num
        acc_ref[1] = acc_ref[1] + cnt

    @pl.when(bidx == _B - 1)
    def _final():
        denom = acc_ref[1] * jnp.float32(_CODE_DIM)
        loss = (1.0 + _COMMITMENT_COST) * acc_ref[0] / denom
        loss_ref[...] = jnp.full((1, 1), loss, jnp.float32)


_stats_call = pl.pallas_call(
    _stats_body,
    grid=(_B,),
    in_specs=[
        pl.BlockSpec((1, _N, _CODE_DIM), lambda b: (b, 0, 0)),
        pl.BlockSpec((1, _N, _CODE_DIM), lambda b: (b, 0, 0)),
    ],
    out_specs=[
        pl.BlockSpec((1, _SUB, 128), lambda b: (b, 0, 0)),
        pl.BlockSpec((1, 1), lambda b: (0, 0)),
    ],
    out_shape=[
        jax.ShapeDtypeStruct((_B, _SUB, 128), jnp.int32),
        jax.ShapeDtypeStruct((1, 1), jnp.float32),
    ],
    scratch_shapes=[pltpu.SMEM((2,), jnp.float32)],
)


def kernel(mlc_proj, code, code_id, W, b):
    idx = code_id.reshape(_NW, _N_CH, _CH).astype(jnp.int32)
    quant_flat = _sc_gather(code, idx)
    out_flat = _mm_call(quant_flat, W, b.reshape(1, _HIDDEN))
    quant = quant_flat.reshape(_B, _N, _CODE_DIM)
    valid3, loss = _stats_call(quant, mlc_proj)
    valid = valid3.reshape(_B, _N) != 0
    return out_flat.reshape(_B, _N, _HIDDEN), valid, loss.reshape(())


# speedup vs baseline: 1.3439x; 1.3439x over previous
import functools

import jax
import jax.numpy as jnp
from jax import lax
from jax.experimental import pallas as pl
from jax.experimental.pallas import tpu as pltpu
from jax.experimental.pallas import tpu_sc as plsc

_B, _N, _CODE_DIM, _K, _HIDDEN = 16, 1024, 256, 8192, 768
_COMMITMENT_COST = 0.25
_THRESHOLD = 0.5

_TOK = _B * _N


_info = plsc.get_sparse_core_info()
_NC, _NS = _info.num_cores, _info.num_subcores
_NW = _NC * _NS
_PER_W = _TOK // _NW
_CH = 128
_N_CH = _PER_W // _CH
_NB = 3


def _make_sc_gather():
    mesh = plsc.VectorSubcoreMesh(core_axis_name="c", subcore_axis_name="s")

    @functools.partial(
        pl.kernel,
        mesh=mesh,
        out_type=jax.ShapeDtypeStruct((_TOK, _CODE_DIM), jnp.float32),
        scratch_types=[
            pltpu.VMEM((_N_CH, _CH), jnp.int32),
            pltpu.VMEM((_CH, _CODE_DIM), jnp.float32),
            pltpu.VMEM((_CH, _CODE_DIM), jnp.float32),
            pltpu.VMEM((_CH, _CODE_DIM), jnp.float32),
            pltpu.SemaphoreType.DMA,
            pltpu.SemaphoreType.DMA,
            pltpu.SemaphoreType.DMA,
            pltpu.SemaphoreType.DMA,
            pltpu.SemaphoreType.DMA,
            pltpu.SemaphoreType.DMA,
        ],
    )
    def sc_gather(table_hbm, idx_hbm, out_hbm, idx_v,
                  rows0, rows1, rows2, g0, g1, g2, w0, w1, w2):
        wid = lax.axis_index("s") * _NC + lax.axis_index("c")
        base = wid * _PER_W
        pltpu.sync_copy(idx_hbm.at[wid], idx_v)
        bufs = (rows0, rows1, rows2)
        gsems = (g0, g1, g2)
        wsems = (w0, w1, w2)
        gcopies = [None] * _NB
        wcopies = [None] * _N_CH
        for c in range(_NB):
            gcopies[c] = pltpu.async_copy(
                table_hbm.at[idx_v.at[c]], bufs[c], gsems[c])
        for c in range(_N_CH):
            bi = c % _NB
            gcopies[bi].wait()
            wcopies[c] = pltpu.async_copy(
                bufs[bi], out_hbm.at[pl.ds(base + c * _CH, _CH)], wsems[bi])
            nc = c + _NB
            if nc < _N_CH:
                wcopies[nc - _NB].wait()
                wcopies[nc - _NB] = None
                gcopies[bi] = pltpu.async_copy(
                    table_hbm.at[idx_v.at[nc]], bufs[bi], gsems[bi])
        for c in range(_N_CH):
            if wcopies[c] is not None:
                wcopies[c].wait()

    return sc_gather


_sc_gather = _make_sc_gather()


_SUB = _N // 128


def _tc_body(q_ref, m_ref, w_ref, b_ref, o_ref, valid_ref, loss_ref, acc_ref):
    bidx = pl.program_id(0)
    q2 = q_ref[0]
    o_ref[0] = lax.dot_general(
        q2, w_ref[...], (((1,), (1,)), ((), ())),
        preferred_element_type=jnp.float32) + b_ref[...]

    q3 = q2.reshape(_SUB, 128, _CODE_DIM)
    m3 = m_ref[0].reshape(_SUB, 128, _CODE_DIM)
    sim = jnp.sum(q3 * m3, axis=2)
    sq = jnp.sum((m3 - q3) ** 2, axis=2)

    neg = jnp.float32(-jnp.inf)
    cur = jnp.float32(jnp.inf)
    removed = jnp.float32(0.0)
    for _ in range(5):
        mmax = jnp.max(jnp.where(sim < cur, sim, neg))
        cnt_eq = jnp.sum(jnp.where(sim == mmax, 1.0, 0.0))
        upd = removed < 5.0
        removed = jnp.where(upd, removed + cnt_eq, removed)
        cur = jnp.where(upd, mmax, cur)

    thresh = jnp.minimum(cur, jnp.float32(_THRESHOLD))
    validf = (sim >= thresh).astype(jnp.float32)
    valid_ref[0] = validf.astype(jnp.int32)

    num = jnp.sum(sq * validf)
    cnt = jnp.sum(validf)

    @pl.when(bidx == 0)
    def _init():
        acc_ref[0] = num
        acc_ref[1] = cnt

    @pl.when(bidx > 0)
    def _accum():
        acc_ref[0] = acc_ref[0] + num
        acc_ref[1] = acc_ref[1] + cnt

    @pl.when(bidx == _B - 1)
    def _final():
        denom = acc_ref[1] * jnp.float32(_CODE_DIM)
        loss = (1.0 + _COMMITMENT_COST) * acc_ref[0] / denom
        loss_ref[...] = jnp.full((1, 1), loss, jnp.float32)


_tc_call = pl.pallas_call(
    _tc_body,
    grid=(_B,),
    in_specs=[
        pl.BlockSpec((1, _N, _CODE_DIM), lambda b: (b, 0, 0)),
        pl.BlockSpec((1, _N, _CODE_DIM), lambda b: (b, 0, 0)),
        pl.BlockSpec((_HIDDEN, _CODE_DIM), lambda b: (0, 0)),
        pl.BlockSpec((1, _HIDDEN), lambda b: (0, 0)),
    ],
    out_specs=[
        pl.BlockSpec((1, _N, _HIDDEN), lambda b: (b, 0, 0)),
        pl.BlockSpec((1, _SUB, 128), lambda b: (b, 0, 0)),
        pl.BlockSpec((1, 1), lambda b: (0, 0)),
    ],
    out_shape=[
        jax.ShapeDtypeStruct((_B, _N, _HIDDEN), jnp.float32),
        jax.ShapeDtypeStruct((_B, _SUB, 128), jnp.int32),
        jax.ShapeDtypeStruct((1, 1), jnp.float32),
    ],
    scratch_shapes=[pltpu.SMEM((2,), jnp.float32)],
)


def kernel(mlc_proj, code, code_id, W, b):
    idx = code_id.reshape(_NW, _N_CH, _CH).astype(jnp.int32)
    quant_flat = _sc_gather(code, idx)
    quant = quant_flat.reshape(_B, _N, _CODE_DIM)
    out, valid3, loss = _tc_call(quant, mlc_proj, W, b.reshape(1, _HIDDEN))
    valid = valid3.reshape(_B, _N) != 0
    return out, valid, loss.reshape(())


# speedup vs baseline: 1.4195x; 1.0563x over previous
import functools

import jax
import jax.numpy as jnp
from jax import lax
from jax.experimental import pallas as pl
from jax.experimental.pallas import tpu as pltpu
from jax.experimental.pallas import tpu_sc as plsc

_B, _N, _CODE_DIM, _K, _HIDDEN = 16, 1024, 256, 8192, 768
_COMMITMENT_COST = 0.25
_THRESHOLD = 0.5

_TOK = _B * _N


_info = plsc.get_sparse_core_info()
_NC, _NS = _info.num_cores, _info.num_subcores
_NW = _NC * _NS
_PER_W = _TOK // _NW
_CH = 128
_N_CH = _PER_W // _CH
_NB = 3


def _make_sc_gather():
    mesh = plsc.VectorSubcoreMesh(core_axis_name="c", subcore_axis_name="s")

    @functools.partial(
        pl.kernel,
        mesh=mesh,
        out_type=jax.ShapeDtypeStruct((_TOK, _CODE_DIM), jnp.float32),
        scratch_types=[
            pltpu.VMEM((_N_CH, _CH), jnp.int32),
            pltpu.VMEM((_CH, _CODE_DIM), jnp.float32),
            pltpu.VMEM((_CH, _CODE_DIM), jnp.float32),
            pltpu.VMEM((_CH, _CODE_DIM), jnp.float32),
            pltpu.SemaphoreType.DMA,
            pltpu.SemaphoreType.DMA,
            pltpu.SemaphoreType.DMA,
            pltpu.SemaphoreType.DMA,
            pltpu.SemaphoreType.DMA,
            pltpu.SemaphoreType.DMA,
        ],
    )
    def sc_gather(table_hbm, idx_hbm, out_hbm, idx_v,
                  rows0, rows1, rows2, g0, g1, g2, w0, w1, w2):
        wid = lax.axis_index("s") * _NC + lax.axis_index("c")
        base = wid * _PER_W
        pltpu.sync_copy(idx_hbm.at[wid], idx_v)
        bufs = (rows0, rows1, rows2)
        gsems = (g0, g1, g2)
        wsems = (w0, w1, w2)
        gcopies = [None] * _NB
        wcopies = [None] * _N_CH
        for c in range(_NB):
            gcopies[c] = pltpu.async_copy(
                table_hbm.at[idx_v.at[c]], bufs[c], gsems[c])
        for c in range(_N_CH):
            bi = c % _NB
            gcopies[bi].wait()
            wcopies[c] = pltpu.async_copy(
                bufs[bi], out_hbm.at[pl.ds(base + c * _CH, _CH)], wsems[bi])
            nc = c + _NB
            if nc < _N_CH:
                wcopies[nc - _NB].wait()
                wcopies[nc - _NB] = None
                gcopies[bi] = pltpu.async_copy(
                    table_hbm.at[idx_v.at[nc]], bufs[bi], gsems[bi])
        for c in range(_N_CH):
            if wcopies[c] is not None:
                wcopies[c].wait()

    return sc_gather


_sc_gather = _make_sc_gather()


_SUB = _N // 128
_BB = 2
_STEPS = _B // _BB


def _tc_body(q_ref, m_ref, w_ref, b_ref, o_ref, valid_ref, loss_ref, acc_ref):
    bidx = pl.program_id(0)
    q2 = q_ref[...].reshape(_BB * _N, _CODE_DIM)
    o_ref[...] = (lax.dot_general(
        q2, w_ref[...], (((1,), (1,)), ((), ())),
        preferred_element_type=jnp.float32) + b_ref[...]).reshape(
            _BB, _N, _HIDDEN)

    q4 = q_ref[...].reshape(_BB, _SUB, 128, _CODE_DIM)
    m4 = m_ref[...].reshape(_BB, _SUB, 128, _CODE_DIM)
    num = jnp.float32(0.0)
    cnt = jnp.float32(0.0)
    for r in range(_BB):
        sim = jnp.sum(q4[r] * m4[r], axis=2)
        sq = jnp.sum((m4[r] - q4[r]) ** 2, axis=2)

        neg = jnp.float32(-jnp.inf)
        cur = jnp.float32(jnp.inf)
        removed = jnp.float32(0.0)
        for _ in range(5):
            mmax = jnp.max(jnp.where(sim < cur, sim, neg))
            cnt_eq = jnp.sum(jnp.where(sim == mmax, 1.0, 0.0))
            upd = removed < 5.0
            removed = jnp.where(upd, removed + cnt_eq, removed)
            cur = jnp.where(upd, mmax, cur)

        thresh = jnp.minimum(cur, jnp.float32(_THRESHOLD))
        validf = (sim >= thresh).astype(jnp.float32)
        valid_ref[r] = validf.astype(jnp.int32)

        num = num + jnp.sum(sq * validf)
        cnt = cnt + jnp.sum(validf)

    @pl.when(bidx == 0)
    def _init():
        acc_ref[0] = num
        acc_ref[1] = cnt

    @pl.when(bidx > 0)
    def _accum():
        acc_ref[0] = acc_ref[0] + num
        acc_ref[1] = acc_ref[1] + cnt

    @pl.when(bidx == _STEPS - 1)
    def _final():
        denom = acc_ref[1] * jnp.float32(_CODE_DIM)
        loss = (1.0 + _COMMITMENT_COST) * acc_ref[0] / denom
        loss_ref[...] = jnp.full((1, 1), loss, jnp.float32)


_tc_call = pl.pallas_call(
    _tc_body,
    grid=(_STEPS,),
    in_specs=[
        pl.BlockSpec((_BB, _N, _CODE_DIM), lambda b: (b, 0, 0)),
        pl.BlockSpec((_BB, _N, _CODE_DIM), lambda b: (b, 0, 0)),
        pl.BlockSpec((_HIDDEN, _CODE_DIM), lambda b: (0, 0)),
        pl.BlockSpec((1, _HIDDEN), lambda b: (0, 0)),
    ],
    out_specs=[
        pl.BlockSpec((_BB, _N, _HIDDEN), lambda b: (b, 0, 0)),
        pl.BlockSpec((_BB, _SUB, 128), lambda b: (b, 0, 0)),
        pl.BlockSpec((1, 1), lambda b: (0, 0)),
    ],
    out_shape=[
        jax.ShapeDtypeStruct((_B, _N, _HIDDEN), jnp.float32),
        jax.ShapeDtypeStruct((_B, _SUB, 128), jnp.int32),
        jax.ShapeDtypeStruct((1, 1), jnp.float32),
    ],
    scratch_shapes=[pltpu.SMEM((2,), jnp.float32)],
)


def kernel(mlc_proj, code, code_id, W, b):
    idx = code_id.reshape(_NW, _N_CH, _CH).astype(jnp.int32)
    quant_flat = _sc_gather(code, idx)
    quant = quant_flat.reshape(_B, _N, _CODE_DIM)
    out, valid3, loss = _tc_call(quant, mlc_proj, W, b.reshape(1, _HIDDEN))
    valid = valid3.reshape(_B, _N) != 0
    return out, valid, loss.reshape(())
